# Initial kernel scaffold; baseline (speedup 1.0000x reference)
import jax
import jax.numpy as jnp
from jax.experimental import pallas as pl


def kernel(inputs, embedding_matrix):
    raise NotImplementedError("write your pallas kernel here")


# speedup vs baseline: 1.0981x; 1.0981x over previous
import functools

import jax
import jax.numpy as jnp
from jax import lax
from jax.experimental import pallas as pl
from jax.experimental.pallas import tpu as pltpu
from jax.experimental.pallas import tpu_sc as plsc

NUM_CORES = 2
NUM_SUBCORES = 16
NUM_WORKERS = NUM_CORES * NUM_SUBCORES
IDX_W = 128
STREAMS_PER_CHUNK = 10
CHUNK = IDX_W * STREAMS_PER_CHUNK


def _make_gather(total, d_model):
    assert total % (NUM_WORKERS * CHUNK) == 0
    per_worker = total // NUM_WORKERS
    n_chunks = per_worker // CHUNK

    mesh = plsc.VectorSubcoreMesh(core_axis_name="c", subcore_axis_name="s")

    @functools.partial(
        pl.kernel,
        out_type=jax.ShapeDtypeStruct((total, d_model), jnp.float32),
        mesh=mesh,
        compiler_params=pltpu.CompilerParams(use_tc_tiling_on_sc=False),
        scratch_types=[
            pltpu.VMEM((STREAMS_PER_CHUNK, IDX_W), jnp.int32),
            pltpu.VMEM((CHUNK, d_model), jnp.float32),
            pltpu.SemaphoreType.DMA,
        ],
    )
    def gather_kernel(idx_hbm, table_hbm, out_hbm, idx_v, rows_v, sem):
        wid = lax.axis_index("s") * NUM_CORES + lax.axis_index("c")
        base_row = wid * (per_worker // IDX_W)

        def chunk_body(i, carry):
            row0 = base_row + i * STREAMS_PER_CHUNK
            pltpu.sync_copy(idx_hbm.at[pl.ds(row0, STREAMS_PER_CHUNK)], idx_v)
            for j in range(STREAMS_PER_CHUNK):
                pltpu.async_copy(
                    table_hbm.at[idx_v.at[j]],
                    rows_v.at[pl.ds(j * IDX_W, IDX_W)],
                    sem,
                )
            for j in range(STREAMS_PER_CHUNK):
                pltpu.make_async_copy(
                    table_hbm.at[idx_v.at[j]],
                    rows_v.at[pl.ds(j * IDX_W, IDX_W)],
                    sem,
                ).wait()
            pltpu.sync_copy(rows_v, out_hbm.at[pl.ds(row0 * IDX_W, CHUNK)])
            return carry

        lax.fori_loop(0, n_chunks, chunk_body, 0)

    return gather_kernel


def kernel(inputs, embedding_matrix):
    batch, hist = inputs.shape
    total = batch * hist
    d_model = embedding_matrix.shape[1]
    flat_idx = inputs.reshape(total // IDX_W, IDX_W).astype(jnp.int32)
    table = embedding_matrix.astype(jnp.float32)
    out = _make_gather(total, d_model)(flat_idx, table)
    return out.reshape(batch, hist, d_model)


# speedup vs baseline: 1.1078x; 1.0088x over previous
import functools

import jax
import jax.numpy as jnp
from jax import lax
from jax.experimental import pallas as pl
from jax.experimental.pallas import tpu as pltpu
from jax.experimental.pallas import tpu_sc as plsc

NUM_CORES = 2
NUM_SUBCORES = 16
NUM_WORKERS = NUM_CORES * NUM_SUBCORES
IDX_W = 128
STREAMS_PER_CHUNK = 10
CHUNK = IDX_W * STREAMS_PER_CHUNK


def _make_gather(total, d_model):
    assert total % (NUM_WORKERS * 2 * CHUNK) == 0
    per_worker = total // NUM_WORKERS
    n_pairs = per_worker // (2 * CHUNK)

    mesh = plsc.VectorSubcoreMesh(core_axis_name="c", subcore_axis_name="s")

    @functools.partial(
        pl.kernel,
        out_type=jax.ShapeDtypeStruct((total, d_model), jnp.float32),
        mesh=mesh,
        compiler_params=pltpu.CompilerParams(use_tc_tiling_on_sc=False),
        scratch_types=[
            pltpu.VMEM((2, STREAMS_PER_CHUNK, IDX_W), jnp.int32),
            pltpu.VMEM((2, CHUNK, d_model), jnp.float32),
            pltpu.SemaphoreType.DMA,
            pltpu.SemaphoreType.DMA,
        ],
    )
    def gather_kernel(idx_hbm, table_hbm, out_hbm, idx_v, rows_v, sem0, sem1):
        wid = lax.axis_index("s") * NUM_CORES + lax.axis_index("c")
        base_row = wid * (per_worker // IDX_W)

        def stage_and_fire(chunk, slot, sem):
            row0 = base_row + chunk * STREAMS_PER_CHUNK
            pltpu.sync_copy(
                idx_hbm.at[pl.ds(row0, STREAMS_PER_CHUNK)], idx_v.at[slot]
            )
            for j in range(STREAMS_PER_CHUNK):
                pltpu.async_copy(
                    table_hbm.at[idx_v.at[slot, j]],
                    rows_v.at[slot, pl.ds(j * IDX_W, IDX_W)],
                    sem,
                )

        def drain_and_store(chunk, slot, sem):
            row0 = base_row + chunk * STREAMS_PER_CHUNK
            for j in range(STREAMS_PER_CHUNK):
                pltpu.make_async_copy(
                    table_hbm.at[idx_v.at[slot, j]],
                    rows_v.at[slot, pl.ds(j * IDX_W, IDX_W)],
                    sem,
                ).wait()
            pltpu.sync_copy(
                rows_v.at[slot], out_hbm.at[pl.ds(row0 * IDX_W, CHUNK)]
            )

        stage_and_fire(0, 0, sem0)

        def pair_body(g, carry):
            a = 2 * g
            stage_and_fire(a + 1, 1, sem1)
            drain_and_store(a, 0, sem0)
            stage_and_fire(a + 2, 0, sem0)
            drain_and_store(a + 1, 1, sem1)
            return carry

        lax.fori_loop(0, n_pairs - 1, pair_body, 0)

        a_last = 2 * (n_pairs - 1)
        stage_and_fire(a_last + 1, 1, sem1)
        drain_and_store(a_last, 0, sem0)
        drain_and_store(a_last + 1, 1, sem1)

    return gather_kernel


def kernel(inputs, embedding_matrix):
    batch, hist = inputs.shape
    total = batch * hist
    d_model = embedding_matrix.shape[1]
    flat_idx = inputs.reshape(total // IDX_W, IDX_W).astype(jnp.int32)
    table = embedding_matrix.astype(jnp.float32)
    out = _make_gather(total, d_model)(flat_idx, table)
    return out.reshape(batch, hist, d_model)
